# baseline (speedup 1.0000x reference)
import functools
import math

import jax
import jax.numpy as jnp
from jax.experimental import pallas as pl
from jax.experimental.pallas import tpu as pltpu

_RATIO = 0.8
_DEPTH = 3
_BLK = 256
_MMB = (512, 256)


def _pad_up(k):
    p = ((k + 127) // 128) * 128
    while not any(p % b == 0 for b in _MMB):
        p += 128
    return p


def _mm_tile(p):
    return next(b for b in _MMB if p % b == 0)


def _red_blk(p):
    return _BLK if p % _BLK == 0 else 128


def _colsum_diag_kernel(m_ref, cs_ref, dg_ref):
    j = pl.program_id(0)
    bj = m_ref.shape[0]
    n = m_ref.shape[1]

    @pl.when(j == 0)
    def _():
        cs_ref[...] = jnp.zeros_like(cs_ref)
        dg_ref[...] = jnp.zeros_like(dg_ref)

    blk = m_ref[...]
    cs_ref[...] += jnp.sum(blk, axis=0, keepdims=True)
    row = jax.lax.broadcasted_iota(jnp.int32, (bj, n), 0)
    col = jax.lax.broadcasted_iota(jnp.int32, (bj, n), 1)
    mask = (col == row + j * bj).astype(blk.dtype)
    dg_ref[...] += jnp.sum(blk * mask, axis=0, keepdims=True)


def _colsum_diag(m):
    n = m.shape[0]
    blk = _red_blk(n)
    grid = (n // blk,)
    cs, dg = pl.pallas_call(
        _colsum_diag_kernel,
        grid=grid,
        in_specs=[pl.BlockSpec((blk, n), lambda j: (j, 0))],
        out_specs=[
            pl.BlockSpec((1, n), lambda j: (0, 0)),
            pl.BlockSpec((1, n), lambda j: (0, 0)),
        ],
        out_shape=[
            jax.ShapeDtypeStruct((1, n), jnp.float32),
            jax.ShapeDtypeStruct((1, n), jnp.float32),
        ],
    )(m)
    return cs, dg


def _conv_kernel(m_ref, x_ref, res_ref, cs_ref, dg_ref, w_ref, b_ref,
                 out_ref, acc_ref, *, relu, has_res, bj):
    j = pl.program_id(0)
    nsteps = pl.num_programs(0)

    @pl.when(j == 0)
    def _():
        acc_ref[...] = jnp.zeros_like(acc_ref)

    j0 = j * bj
    xb = x_ref[pl.ds(j0, bj), :]
    if has_res:
        xb = xb + res_ref[pl.ds(j0, bj), :]
    zb = jnp.dot(xb, w_ref[...], preferred_element_type=jnp.float32)
    dis_b = jax.lax.rsqrt(cs_ref[pl.ds(j0, bj), :] - dg_ref[pl.ds(j0, bj), :]
                          + 2.0)
    ub = zb * dis_b
    acc_ref[...] += jax.lax.dot_general(
        m_ref[...], ub, (((0,), (0,)), ((), ())),
        preferred_element_type=jnp.float32)

    @pl.when(j == nsteps - 1)
    def _():
        xa = x_ref[...]
        if has_res:
            xa = xa + res_ref[...]
        za = jnp.dot(xa, w_ref[...], preferred_element_type=jnp.float32)
        dis = jax.lax.rsqrt(cs_ref[...] - dg_ref[...] + 2.0)
        self_c = (2.0 - dg_ref[...]) * dis * dis
        out = dis * acc_ref[...] + self_c * za + b_ref[...]
        if relu:
            out = jnp.maximum(out, 0.0)
        out_ref[...] = out


def _gcn_conv(m, cs_col, dg_col, x, w, b, res=None, relu=True):
    n = m.shape[0]
    d_in = x.shape[1]
    d_out = w.shape[1]
    blk = _red_blk(n)
    grid = (n // blk,)
    has_res = res is not None
    if res is None:
        res = jnp.zeros((n, d_in), jnp.float32)
    return pl.pallas_call(
        functools.partial(_conv_kernel, relu=relu, has_res=has_res, bj=blk),
        grid=grid,
        in_specs=[
            pl.BlockSpec((blk, n), lambda j: (j, 0)),
            pl.BlockSpec((n, d_in), lambda j: (0, 0)),
            pl.BlockSpec((n, d_in), lambda j: (0, 0)),
            pl.BlockSpec((n, 1), lambda j: (0, 0)),
            pl.BlockSpec((n, 1), lambda j: (0, 0)),
            pl.BlockSpec((d_in, d_out), lambda j: (0, 0)),
            pl.BlockSpec((1, d_out), lambda j: (0, 0)),
        ],
        out_specs=pl.BlockSpec((n, d_out), lambda j: (0, 0)),
        out_shape=jax.ShapeDtypeStruct((n, d_out), jnp.float32),
        scratch_shapes=[pltpu.VMEM((n, d_out), jnp.float32)],
    )(m, x, res, cs_col, dg_col, w, b)


def _aug_kernel(mr_ref, mtr_ref, pr_ref, pc_ref, out_ref, outt_ref, *,
                bi, bj, bm, k):
    gm = pl.program_id(2)

    @pl.when(gm == 0)
    def _():
        out_ref[...] = jnp.zeros_like(out_ref)

    m0 = gm * bm
    col_r = jax.lax.broadcasted_iota(jnp.int32, (bi, bm), 1) + m0
    mr = mr_ref[...] + jnp.where(col_r == pr_ref[...], 1.0, 0.0)
    col_c = jax.lax.broadcasted_iota(jnp.int32, (bj, bm), 1) + m0
    mtr = mtr_ref[...] + jnp.where(col_c == pc_ref[...], 1.0, 0.0)
    out_ref[...] += jax.lax.dot_general(
        mr, mtr, (((1,), (1,)), ((), ())),
        preferred_element_type=jnp.float32)

    @pl.when(gm == pl.num_programs(2) - 1)
    def _():
        gi = pl.program_id(0)
        gj = pl.program_id(1)
        ri = jax.lax.broadcasted_iota(jnp.int32, (bi, bj), 0) + gi * bi
        cj = jax.lax.broadcasted_iota(jnp.int32, (bi, bj), 1) + gj * bj
        keep = jnp.logical_and(jnp.logical_and(ri != cj, ri < k), cj < k)
        res = jnp.where(keep, out_ref[...], 0.0)
        out_ref[...] = res
        outt_ref[...] = res.T


def _augment_pool(mr, mtr, perm_p, k):
    p, n = mr.shape
    bi = bj = _mm_tile(p)
    bm = next(b for b in (1024, 896, 768, 640, 512, 448, 384, 320, 256, 128)
              if n % b == 0)
    pr = perm_p.reshape(p, 1)
    grid = (p // bi, p // bj, n // bm)
    return pl.pallas_call(
        functools.partial(_aug_kernel, bi=bi, bj=bj, bm=bm, k=k),
        grid=grid,
        in_specs=[
            pl.BlockSpec((bi, bm), lambda i, j, m: (i, m)),
            pl.BlockSpec((bj, bm), lambda i, j, m: (j, m)),
            pl.BlockSpec((bi, 1), lambda i, j, m: (i, 0)),
            pl.BlockSpec((bj, 1), lambda i, j, m: (j, 0)),
        ],
        out_specs=[
            pl.BlockSpec((bi, bj), lambda i, j, m: (i, j)),
            pl.BlockSpec((bj, bi), lambda i, j, m: (j, i)),
        ],
        out_shape=[
            jax.ShapeDtypeStruct((p, p), jnp.float32),
            jax.ShapeDtypeStruct((p, p), jnp.float32),
        ],
        compiler_params=pltpu.CompilerParams(
            dimension_semantics=("parallel", "parallel", "arbitrary")),
    )(mr, mtr, pr, pr)


def kernel(x, edge_index, batch, params):
    n = x.shape[0]
    down_W, down_b = params["down_W"], params["down_b"]
    pool_w, up_W, up_b = params["pool_w"], params["up_W"], params["up_b"]

    src, dst = edge_index[0], edge_index[1]
    w_e = jnp.where(src == dst, 0.0, 1.0)
    a = jnp.zeros((n, n), jnp.float32).at[src, dst].add(w_e)
    at = a.T

    def deg(m):
        cs, dg = _colsum_diag(m)
        return cs.T, dg.T

    cs0, dg0 = deg(a)
    xh = _gcn_conv(a, cs0, dg0, x, down_W[0],
                   down_b[0].reshape(1, -1), relu=True)

    sizes = [n]
    for _ in range(_DEPTH):
        sizes.append(int(math.ceil(_RATIO * sizes[-1])))
    pads = [n] + [_pad_up(k) for k in sizes[1:]]

    xs = [xh]
    adjs = [a]
    degs = [(cs0, dg0)]
    perms = []

    m_cur, mt_cur, x_cur = a, at, xh
    for lvl in range(1, _DEPTH + 1):
        k_prev, k = sizes[lvl - 1], sizes[lvl]
        p_prev, p = pads[lvl - 1], pads[lvl]
        w = pool_w[lvl - 1]
        score = jnp.tanh((x_cur[:k_prev] @ w) / jnp.linalg.norm(w))
        vals, perm = jax.lax.top_k(score, k)
        perms.append((perm, vals, k))

        pad_r = -jnp.ones((p - k,), jnp.int32)
        perm_p = jnp.concatenate([perm, pad_r])
        mr = m_cur[perm_p, :]
        mtr = mt_cur[perm_p, :]
        m_cur, mt_cur = _augment_pool(mr, mtr, perm_p, k)

        x_pool = jnp.zeros((p, x_cur.shape[1]), jnp.float32).at[:k].set(
            x_cur[perm] * vals[:, None])

        cs, dg = deg(m_cur)
        x_cur = _gcn_conv(m_cur, cs, dg, x_pool, down_W[lvl],
                          down_b[lvl].reshape(1, -1), relu=True)
        if lvl < _DEPTH:
            xs.append(x_cur)
            adjs.append(m_cur)
            degs.append((cs, dg))

    for i in range(_DEPTH):
        j = _DEPTH - 1 - i
        res, mj, (csj, dgj) = xs[j], adjs[j], degs[j]
        perm, _, k = perms[j]
        p_prev = pads[j]
        up = jnp.zeros((p_prev, x_cur.shape[1]), jnp.float32).at[perm].set(
            x_cur[:k])
        x_cur = _gcn_conv(mj, csj, dgj, up, up_W[i],
                          up_b[i].reshape(1, -1), res=res,
                          relu=(i < _DEPTH - 1))

    return x_cur


# speedup vs baseline: 1.9123x; 1.9123x over previous
import functools
import math

import jax
import jax.numpy as jnp
from jax.experimental import pallas as pl
from jax.experimental.pallas import tpu as pltpu

_RATIO = 0.8
_DEPTH = 3
_BLK = 256
_MMB = (512,)


def _pad_up(k):
    return ((k + 511) // 512) * 512


def _mm_tile(p):
    return next(b for b in _MMB if p % b == 0)


def _red_blk(p):
    return _BLK if p % _BLK == 0 else 128


def _colsum_diag_kernel(m_ref, cs_ref, dg_ref):
    j = pl.program_id(0)
    bj = m_ref.shape[0]
    n = m_ref.shape[1]

    @pl.when(j == 0)
    def _():
        cs_ref[...] = jnp.zeros_like(cs_ref)
        dg_ref[...] = jnp.zeros_like(dg_ref)

    blk = m_ref[...]
    cs_ref[...] += jnp.sum(blk, axis=0, keepdims=True)
    row = jax.lax.broadcasted_iota(jnp.int32, (bj, n), 0)
    col = jax.lax.broadcasted_iota(jnp.int32, (bj, n), 1)
    mask = (col == row + j * bj).astype(blk.dtype)
    dg_ref[...] += jnp.sum(blk * mask, axis=0, keepdims=True)


def _colsum_diag(m):
    n = m.shape[0]
    blk = _red_blk(n)
    grid = (n // blk,)
    cs, dg = pl.pallas_call(
        _colsum_diag_kernel,
        grid=grid,
        in_specs=[pl.BlockSpec((blk, n), lambda j: (j, 0))],
        out_specs=[
            pl.BlockSpec((1, n), lambda j: (0, 0)),
            pl.BlockSpec((1, n), lambda j: (0, 0)),
        ],
        out_shape=[
            jax.ShapeDtypeStruct((1, n), jnp.float32),
            jax.ShapeDtypeStruct((1, n), jnp.float32),
        ],
    )(m)
    return cs, dg


def _conv_kernel(m_ref, x_ref, res_ref, cs_ref, dg_ref, w_ref, b_ref,
                 out_ref, acc_ref, *, relu, has_res, bj):
    j = pl.program_id(0)
    nsteps = pl.num_programs(0)

    @pl.when(j == 0)
    def _():
        acc_ref[...] = jnp.zeros_like(acc_ref)

    j0 = j * bj
    xb = x_ref[pl.ds(j0, bj), :]
    if has_res:
        xb = xb + res_ref[pl.ds(j0, bj), :]
    zb = jnp.dot(xb, w_ref[...], preferred_element_type=jnp.float32)
    dis_b = jax.lax.rsqrt(cs_ref[pl.ds(j0, bj), :] - dg_ref[pl.ds(j0, bj), :]
                          + 2.0)
    ub = zb * dis_b
    acc_ref[...] += jax.lax.dot_general(
        m_ref[...], ub, (((0,), (0,)), ((), ())),
        preferred_element_type=jnp.float32)

    @pl.when(j == nsteps - 1)
    def _():
        xa = x_ref[...]
        if has_res:
            xa = xa + res_ref[...]
        za = jnp.dot(xa, w_ref[...], preferred_element_type=jnp.float32)
        dis = jax.lax.rsqrt(cs_ref[...] - dg_ref[...] + 2.0)
        self_c = (2.0 - dg_ref[...]) * dis * dis
        out = dis * acc_ref[...] + self_c * za + b_ref[...]
        if relu:
            out = jnp.maximum(out, 0.0)
        out_ref[...] = out


def _gcn_conv(m, cs_col, dg_col, x, w, b, res=None, relu=True):
    n = m.shape[0]
    d_in = x.shape[1]
    d_out = w.shape[1]
    blk = _red_blk(n)
    grid = (n // blk,)
    has_res = res is not None
    if res is None:
        res = jnp.zeros((n, d_in), jnp.float32)
    return pl.pallas_call(
        functools.partial(_conv_kernel, relu=relu, has_res=has_res, bj=blk),
        grid=grid,
        in_specs=[
            pl.BlockSpec((blk, n), lambda j: (j, 0)),
            pl.BlockSpec((n, d_in), lambda j: (0, 0)),
            pl.BlockSpec((n, d_in), lambda j: (0, 0)),
            pl.BlockSpec((n, 1), lambda j: (0, 0)),
            pl.BlockSpec((n, 1), lambda j: (0, 0)),
            pl.BlockSpec((d_in, d_out), lambda j: (0, 0)),
            pl.BlockSpec((1, d_out), lambda j: (0, 0)),
        ],
        out_specs=pl.BlockSpec((n, d_out), lambda j: (0, 0)),
        out_shape=jax.ShapeDtypeStruct((n, d_out), jnp.float32),
        scratch_shapes=[pltpu.VMEM((n, d_out), jnp.float32)],
    )(m, x, res, cs_col, dg_col, w, b)


def _aug_kernel(mr_ref, mtr_ref, pr_ref, pc_ref, out_ref, outt_ref, *,
                bi, bj, bm, k):
    gm = pl.program_id(2)

    @pl.when(gm == 0)
    def _():
        out_ref[...] = jnp.zeros_like(out_ref)

    m0 = gm * bm
    col_r = jax.lax.broadcasted_iota(jnp.int32, (bi, bm), 1) + m0
    mr = mr_ref[...] + jnp.where(col_r == pr_ref[...], 1.0, 0.0)
    col_c = jax.lax.broadcasted_iota(jnp.int32, (bj, bm), 1) + m0
    mtr = mtr_ref[...] + jnp.where(col_c == pc_ref[...], 1.0, 0.0)
    out_ref[...] += jax.lax.dot_general(
        mr, mtr, (((1,), (1,)), ((), ())),
        preferred_element_type=jnp.float32)

    @pl.when(gm == pl.num_programs(2) - 1)
    def _():
        gi = pl.program_id(0)
        gj = pl.program_id(1)
        ri = jax.lax.broadcasted_iota(jnp.int32, (bi, bj), 0) + gi * bi
        cj = jax.lax.broadcasted_iota(jnp.int32, (bi, bj), 1) + gj * bj
        keep = jnp.logical_and(jnp.logical_and(ri != cj, ri < k), cj < k)
        res = jnp.where(keep, out_ref[...], 0.0)
        out_ref[...] = res
        outt_ref[...] = res.T


def _augment_pool(mr, mtr, perm_p, k):
    p, n = mr.shape
    bi = bj = _mm_tile(p)
    bm = next(b for b in (1024, 896, 768, 640, 512, 448, 384, 320, 256, 128)
              if n % b == 0)
    pr = perm_p.reshape(p, 1)
    grid = (p // bi, p // bj, n // bm)
    return pl.pallas_call(
        functools.partial(_aug_kernel, bi=bi, bj=bj, bm=bm, k=k),
        grid=grid,
        in_specs=[
            pl.BlockSpec((bi, bm), lambda i, j, m: (i, m)),
            pl.BlockSpec((bj, bm), lambda i, j, m: (j, m)),
            pl.BlockSpec((bi, 1), lambda i, j, m: (i, 0)),
            pl.BlockSpec((bj, 1), lambda i, j, m: (j, 0)),
        ],
        out_specs=[
            pl.BlockSpec((bi, bj), lambda i, j, m: (i, j)),
            pl.BlockSpec((bj, bi), lambda i, j, m: (j, i)),
        ],
        out_shape=[
            jax.ShapeDtypeStruct((p, p), jnp.float32),
            jax.ShapeDtypeStruct((p, p), jnp.float32),
        ],
        compiler_params=pltpu.CompilerParams(
            dimension_semantics=("parallel", "parallel", "arbitrary")),
    )(mr, mtr, pr, pr)


def kernel(x, edge_index, batch, params):
    n = x.shape[0]
    down_W, down_b = params["down_W"], params["down_b"]
    pool_w, up_W, up_b = params["pool_w"], params["up_W"], params["up_b"]

    src, dst = edge_index[0], edge_index[1]
    w_e = jnp.where(src == dst, 0.0, 1.0)
    a = jnp.zeros((n, n), jnp.float32).at[src, dst].add(w_e)
    at = a.T

    def deg(m):
        cs, dg = _colsum_diag(m)
        return cs.T, dg.T

    cs0, dg0 = deg(a)
    xh = _gcn_conv(a, cs0, dg0, x, down_W[0],
                   down_b[0].reshape(1, -1), relu=True)

    sizes = [n]
    for _ in range(_DEPTH):
        sizes.append(int(math.ceil(_RATIO * sizes[-1])))
    pads = [n] + [_pad_up(k) for k in sizes[1:]]

    xs = [xh]
    adjs = [a]
    degs = [(cs0, dg0)]
    perms = []

    m_cur, mt_cur, x_cur = a, at, xh
    for lvl in range(1, _DEPTH + 1):
        k_prev, k = sizes[lvl - 1], sizes[lvl]
        p_prev, p = pads[lvl - 1], pads[lvl]
        w = pool_w[lvl - 1]
        score = jnp.tanh((x_cur[:k_prev] @ w) / jnp.linalg.norm(w))
        vals, perm = jax.lax.top_k(score, k)
        perms.append((perm, vals, k))

        pad_r = -jnp.ones((p - k,), jnp.int32)
        perm_p = jnp.concatenate([perm, pad_r])
        mr = m_cur[perm_p, :]
        mtr = mt_cur[perm_p, :]
        m_cur, mt_cur = _augment_pool(mr, mtr, perm_p, k)

        x_pool = jnp.zeros((p, x_cur.shape[1]), jnp.float32).at[:k].set(
            x_cur[perm] * vals[:, None])

        cs, dg = deg(m_cur)
        x_cur = _gcn_conv(m_cur, cs, dg, x_pool, down_W[lvl],
                          down_b[lvl].reshape(1, -1), relu=True)
        if lvl < _DEPTH:
            xs.append(x_cur)
            adjs.append(m_cur)
            degs.append((cs, dg))

    for i in range(_DEPTH):
        j = _DEPTH - 1 - i
        res, mj, (csj, dgj) = xs[j], adjs[j], degs[j]
        perm, _, k = perms[j]
        p_prev = pads[j]
        up = jnp.zeros((p_prev, x_cur.shape[1]), jnp.float32).at[perm].set(
            x_cur[:k])
        x_cur = _gcn_conv(mj, csj, dgj, up, up_W[i],
                          up_b[i].reshape(1, -1), res=res,
                          relu=(i < _DEPTH - 1))

    return x_cur
